# Initial kernel scaffold; baseline (speedup 1.0000x reference)
import jax
import jax.numpy as jnp
from jax.experimental import pallas as pl


def kernel(x, pos_table):
    raise NotImplementedError("write your pallas kernel here")


# speedup vs baseline: 1.6820x; 1.6820x over previous
import jax
import jax.numpy as jnp
from jax.experimental import pallas as pl

BLOCK_L = 512


def _add_kernel(x_ref, pos_ref, o_ref):
    o_ref[...] = x_ref[...] + pos_ref[...]


def kernel(x, pos_table):
    B, L, D = x.shape
    grid = (L // BLOCK_L, B)
    return pl.pallas_call(
        _add_kernel,
        grid=grid,
        in_specs=[
            pl.BlockSpec((1, BLOCK_L, D), lambda l, b: (b, l, 0)),
            pl.BlockSpec((BLOCK_L, D), lambda l, b: (l, 0)),
        ],
        out_specs=pl.BlockSpec((1, BLOCK_L, D), lambda l, b: (b, l, 0)),
        out_shape=jax.ShapeDtypeStruct((B, L, D), x.dtype),
    )(x, pos_table)


# speedup vs baseline: 1.8542x; 1.1024x over previous
import jax
import jax.numpy as jnp
from jax.experimental import pallas as pl

BLOCK_L = 1024


def _add_kernel(x_ref, pos_ref, o_ref):
    o_ref[...] = x_ref[...] + pos_ref[...]


def kernel(x, pos_table):
    B, L, D = x.shape
    grid = (L // BLOCK_L, B)
    return pl.pallas_call(
        _add_kernel,
        grid=grid,
        in_specs=[
            pl.BlockSpec((1, BLOCK_L, D), lambda l, b: (b, l, 0)),
            pl.BlockSpec((BLOCK_L, D), lambda l, b: (l, 0)),
        ],
        out_specs=pl.BlockSpec((1, BLOCK_L, D), lambda l, b: (b, l, 0)),
        out_shape=jax.ShapeDtypeStruct((B, L, D), x.dtype),
    )(x, pos_table)


# speedup vs baseline: 1.9662x; 1.0604x over previous
import jax
import jax.numpy as jnp
from jax.experimental import pallas as pl

BLOCK_L = 2048


def _add_kernel(x_ref, pos_ref, o_ref):
    o_ref[...] = x_ref[...] + pos_ref[...]


def kernel(x, pos_table):
    B, L, D = x.shape
    grid = (L // BLOCK_L, B)
    return pl.pallas_call(
        _add_kernel,
        grid=grid,
        in_specs=[
            pl.BlockSpec((1, BLOCK_L, D), lambda l, b: (b, l, 0)),
            pl.BlockSpec((BLOCK_L, D), lambda l, b: (l, 0)),
        ],
        out_specs=pl.BlockSpec((1, BLOCK_L, D), lambda l, b: (b, l, 0)),
        out_shape=jax.ShapeDtypeStruct((B, L, D), x.dtype),
    )(x, pos_table)
